# baseline (speedup 1.0000x reference)
import functools

import jax
import jax.numpy as jnp
from jax import lax
from jax.experimental import pallas as pl
from jax.experimental.pallas import tpu as pltpu
from jax.experimental.pallas import tpu_sc as plsc

VOCAB = 50257
HID = 2048
B = 4
S = 2048
NTOK = B * S

_info = plsc.get_sparse_core_info()
NC = _info.num_cores
NS = _info.num_subcores
NW = NC * NS
TPW = NTOK // NW
C = 8
G = TPW // C
VPR = HID // 16
NBUF = 3


def _make_kernel():
    mesh = plsc.VectorSubcoreMesh(core_axis_name="c", subcore_axis_name="s")

    @functools.partial(
        pl.kernel,
        mesh=mesh,
        out_type=jax.ShapeDtypeStruct((S, B, HID), jnp.float32),
        scratch_types=[
            pltpu.VMEM((G, C), jnp.int32),
            pltpu.VMEM((G, C), jnp.int32),
        ] + [pltpu.VMEM((C, HID), jnp.float32)] * (2 * NBUF)
          + [pltpu.SemaphoreType.DMA] * (3 * NBUF),
    )
    def emb_kernel(wids_hbm, pids_hbm, wtab_hbm, ptab_hbm, out_hbm,
                   widx_v, pidx_v,
                   wbuf0, wbuf1, wbuf2, pbuf0, pbuf1, pbuf2,
                   wsem0, wsem1, wsem2, psem0, psem1, psem2,
                   osem0, osem1, osem2):
        wbufs = (wbuf0, wbuf1, wbuf2)
        pbufs = (pbuf0, pbuf1, pbuf2)
        wsems = (wsem0, wsem1, wsem2)
        psems = (psem0, psem1, psem2)
        osems = (osem0, osem1, osem2)

        wid = lax.axis_index("s") * NC + lax.axis_index("c")
        sbase = wid * (TPW // B)
        pltpu.sync_copy(wids_hbm.at[wid], widx_v)
        pltpu.sync_copy(pids_hbm.at[wid], pidx_v)

        def start_gather(g, b):
            pltpu.async_copy(wtab_hbm.at[widx_v.at[g]], wbufs[b], wsems[b])
            pltpu.async_copy(ptab_hbm.at[pidx_v.at[g]], pbufs[b], psems[b])

        def wait_gather(b):
            pltpu.make_async_copy(
                wtab_hbm.at[pl.ds(0, C)], wbufs[b], wsems[b]).wait()
            pltpu.make_async_copy(
                ptab_hbm.at[pl.ds(0, C)], pbufs[b], psems[b]).wait()

        def wait_store(b):
            for h in range(C // B):
                pltpu.make_async_copy(
                    wbufs[b].at[pl.ds(h * B, B)], out_hbm.at[0],
                    osems[b]).wait()

        def do_add(b):
            def add_body(j, carry):
                col = j * 64
                for r in range(C):
                    for k in range(4):
                        cc = col + k * 16
                        wbufs[b][r, pl.ds(cc, 16)] = (
                            wbufs[b][r, pl.ds(cc, 16)]
                            + pbufs[b][r, pl.ds(cc, 16)]
                        )
                return carry
            lax.fori_loop(0, VPR // 4, add_body, 0)

        def start_store(g, b):
            for h in range(C // B):
                pltpu.async_copy(
                    wbufs[b].at[pl.ds(h * B, B)],
                    out_hbm.at[sbase + g * (C // B) + h],
                    osems[b])

        def slot(g, steady):
            b0 = g % NBUF
            bnext = (g + 2) % NBUF
            if steady or g >= 1:
                wait_store(bnext)
            if steady or g + 2 < G:
                start_gather(g + 2, bnext)
            wait_gather(b0)
            do_add(b0)
            start_store(g, b0)

        start_gather(0, 0)
        start_gather(1, 1)

        for g in range(NBUF):
            slot(g, steady=False)

        def step(i, carry):
            for b in range(NBUF):
                g = i * NBUF + b
                wait_store((b + 2) % NBUF)
                start_gather(g + 2, (b + 2) % NBUF)
                wait_gather(b)
                do_add(b)
                start_store(g, b)
            return carry

        lax.fori_loop(1, G // NBUF, step, 0)

        for g in range(NBUF * (G // NBUF) + NBUF - NBUF, G):
            b0 = g % NBUF
            wait_store((g + 2) % NBUF)
            wait_gather(b0)
            do_add(b0)
            start_store(g, b0)

        wait_store((G - 1) % NBUF)

    return emb_kernel


def kernel(input_ids, position_ids, word_embeddings, position_embeddings):
    ids = jnp.transpose(input_ids.astype(jnp.int32), (1, 0)).reshape(NW, G, C)
    pids = jnp.transpose(position_ids.astype(jnp.int32), (1, 0)).reshape(
        NW, G, C)
    return _make_kernel()(ids, pids, word_embeddings, position_embeddings)


# speedup vs baseline: 1.0656x; 1.0656x over previous
import functools

import jax
import jax.numpy as jnp
from jax import lax
from jax.experimental import pallas as pl
from jax.experimental.pallas import tpu as pltpu
from jax.experimental.pallas import tpu_sc as plsc

VOCAB = 50257
HID = 2048
B = 4
S = 2048
NTOK = B * S

_info = plsc.get_sparse_core_info()
NC = _info.num_cores
NS = _info.num_subcores
NW = NC * NS
TPW = NTOK // NW
C = B
G = TPW // C
VPR = HID // 16
NG = 4
NO = 3
PF = 3


def _make_kernel():
    mesh = plsc.VectorSubcoreMesh(core_axis_name="c", subcore_axis_name="s")

    @functools.partial(
        pl.kernel,
        mesh=mesh,
        out_type=jax.ShapeDtypeStruct((S, B, HID), jnp.float32),
        scratch_types=[
            pltpu.VMEM((G, C), jnp.int32),
            pltpu.VMEM((G, C), jnp.int32),
        ] + [pltpu.VMEM((C, HID), jnp.float32)] * (2 * NG + NO)
          + [pltpu.SemaphoreType.DMA] * (2 * NG + NO),
    )
    def emb_kernel(wids_hbm, pids_hbm, wtab_hbm, ptab_hbm, out_hbm,
                   widx_v, pidx_v,
                   wbuf0, wbuf1, wbuf2, wbuf3,
                   pbuf0, pbuf1, pbuf2, pbuf3,
                   obuf0, obuf1, obuf2,
                   wsem0, wsem1, wsem2, wsem3,
                   psem0, psem1, psem2, psem3,
                   osem0, osem1, osem2):
        wbufs = (wbuf0, wbuf1, wbuf2, wbuf3)
        pbufs = (pbuf0, pbuf1, pbuf2, pbuf3)
        obufs = (obuf0, obuf1, obuf2)
        wsems = (wsem0, wsem1, wsem2, wsem3)
        psems = (psem0, psem1, psem2, psem3)
        osems = (osem0, osem1, osem2)

        wid = lax.axis_index("s") * NC + lax.axis_index("c")
        sbase = wid * (TPW // B)
        pltpu.sync_copy(wids_hbm.at[wid], widx_v)
        pltpu.sync_copy(pids_hbm.at[wid], pidx_v)

        def start_gather(g, b):
            pltpu.async_copy(wtab_hbm.at[widx_v.at[g]], wbufs[b], wsems[b])
            pltpu.async_copy(ptab_hbm.at[pidx_v.at[g]], pbufs[b], psems[b])

        def wait_gather(b):
            pltpu.make_async_copy(
                wtab_hbm.at[pl.ds(0, C)], wbufs[b], wsems[b]).wait()
            pltpu.make_async_copy(
                ptab_hbm.at[pl.ds(0, C)], pbufs[b], psems[b]).wait()

        def wait_store(ob):
            pltpu.make_async_copy(
                obufs[ob], out_hbm.at[0], osems[ob]).wait()

        def do_add(b, ob):
            def add_body(j, carry):
                col = j * 64
                for r in range(C):
                    for k in range(4):
                        cc = col + k * 16
                        obufs[ob][r, pl.ds(cc, 16)] = (
                            wbufs[b][r, pl.ds(cc, 16)]
                            + pbufs[b][r, pl.ds(cc, 16)]
                        )
                return carry
            lax.fori_loop(0, VPR // 4, add_body, 0)

        def start_store(g, ob):
            pltpu.async_copy(obufs[ob], out_hbm.at[sbase + g], osems[ob])

        def slot(g):
            b = g % NG
            ob = g % NO
            if g + PF < G:
                start_gather(g + PF, (g + PF) % NG)
            wait_gather(b)
            if g >= NO:
                wait_store(ob)
            do_add(b, ob)
            start_store(g, ob)

        for g in range(PF):
            start_gather(g, g)

        UNROLL = 12
        for g in range(UNROLL):
            slot(g)

        def step(i, carry):
            for b in range(UNROLL):
                g = i * UNROLL + b
                start_gather(g + PF, (b + PF) % NG)
                wait_gather(b % NG)
                wait_store(b % NO)
                do_add(b % NG, b % NO)
                start_store(g, b % NO)
            return carry

        lax.fori_loop(1, G // UNROLL - 1, step, 0)

        for g in range(G - UNROLL - (G - UNROLL * (G // UNROLL)), G):
            slot(g)

        for ob in range(NO):
            wait_store((G - NO + 1 + ob) % NO)

    return emb_kernel


def kernel(input_ids, position_ids, word_embeddings, position_embeddings):
    ids = jnp.transpose(input_ids.astype(jnp.int32), (1, 0)).reshape(NW, G, C)
    pids = jnp.transpose(position_ids.astype(jnp.int32), (1, 0)).reshape(
        NW, G, C)
    return _make_kernel()(ids, pids, word_embeddings, position_embeddings)


# speedup vs baseline: 1.3369x; 1.2547x over previous
import functools

import jax
import jax.numpy as jnp
from jax import lax
from jax.experimental import pallas as pl
from jax.experimental.pallas import tpu as pltpu
from jax.experimental.pallas import tpu_sc as plsc

VOCAB = 50257
HID = 2048
B = 4
S = 2048
NTOK = B * S

_info = plsc.get_sparse_core_info()
NC = _info.num_cores
NS = _info.num_subcores
NW = NC * NS
TPW = NTOK // NW
C = 8
G = TPW // C
VPR = HID // 16
NWB = 3
NOB = 4
PF = 2
UNROLL = 12


def _make_kernel():
    mesh = plsc.VectorSubcoreMesh(core_axis_name="c", subcore_axis_name="s")

    @functools.partial(
        pl.kernel,
        mesh=mesh,
        out_type=jax.ShapeDtypeStruct((S, B, HID), jnp.float32),
        scratch_types=[
            pltpu.VMEM((G, C), jnp.int32),
            pltpu.VMEM((G, C), jnp.int32),
        ] + [pltpu.VMEM((C, HID), jnp.float32)] * (NWB + NOB)
          + [pltpu.SemaphoreType.DMA] * (NWB + 2 * NOB),
    )
    def emb_kernel(wids_hbm, pids_hbm, wtab_hbm, ptab_hbm, out_hbm,
                   widx_v, pidx_v,
                   wbuf0, wbuf1, wbuf2,
                   obuf0, obuf1, obuf2, obuf3,
                   wsem0, wsem1, wsem2,
                   psem0, psem1, psem2, psem3,
                   osem0, osem1, osem2, osem3):
        wbufs = (wbuf0, wbuf1, wbuf2)
        obufs = (obuf0, obuf1, obuf2, obuf3)
        wsems = (wsem0, wsem1, wsem2)
        psems = (psem0, psem1, psem2, psem3)
        osems = (osem0, osem1, osem2, osem3)

        wid = lax.axis_index("s") * NC + lax.axis_index("c")
        sbase = wid * (TPW // B)
        pltpu.sync_copy(wids_hbm.at[wid], widx_v)
        pltpu.sync_copy(pids_hbm.at[wid], pidx_v)

        def start_gather(g, wb, ob):
            pltpu.async_copy(wtab_hbm.at[widx_v.at[g]], wbufs[wb], wsems[wb])
            pltpu.async_copy(ptab_hbm.at[pidx_v.at[g]], obufs[ob], psems[ob])

        def wait_gather(wb, ob):
            pltpu.make_async_copy(
                wtab_hbm.at[pl.ds(0, C)], wbufs[wb], wsems[wb]).wait()
            pltpu.make_async_copy(
                ptab_hbm.at[pl.ds(0, C)], obufs[ob], psems[ob]).wait()

        def wait_store(ob):
            for h in range(C // B):
                pltpu.make_async_copy(
                    obufs[ob].at[pl.ds(h * B, B)], out_hbm.at[0],
                    osems[ob]).wait()

        def do_add(wb, ob):
            def add_body(j, carry):
                col = j * 32
                for r in range(C):
                    for k in range(2):
                        cc = col + k * 16
                        obufs[ob][r, pl.ds(cc, 16)] = (
                            obufs[ob][r, pl.ds(cc, 16)]
                            + wbufs[wb][r, pl.ds(cc, 16)]
                        )
                return carry
            lax.fori_loop(0, VPR // 2, add_body, 0)

        def start_store(g, ob):
            for h in range(C // B):
                pltpu.async_copy(
                    obufs[ob].at[pl.ds(h * B, B)],
                    out_hbm.at[sbase + g * (C // B) + h],
                    osems[ob])

        def slot(g):
            wb = g % NWB
            ob = g % NOB
            if g + PF < G:
                if g >= PF:
                    wait_store((g + PF) % NOB)
                start_gather(g + PF, (g + PF) % NWB, (g + PF) % NOB)
            wait_gather(wb, ob)
            do_add(wb, ob)
            start_store(g, ob)

        for g in range(PF):
            start_gather(g, g % NWB, g % NOB)

        for g in range(UNROLL):
            slot(g)

        def step(i, carry):
            for b in range(UNROLL):
                g = i * UNROLL + b
                wait_store((b + PF) % NOB)
                start_gather(g + PF, (b + PF) % NWB, (b + PF) % NOB)
                wait_gather(b % NWB, b % NOB)
                do_add(b % NWB, b % NOB)
                start_store(g, b % NOB)
            return carry

        lax.fori_loop(1, (G - UNROLL) // UNROLL, step, 0)

        for g in range(UNROLL + UNROLL * ((G - UNROLL) // UNROLL - 1), G):
            slot(g)

        for g in range(G - PF, G):
            wait_store(g % NOB)

    return emb_kernel


def kernel(input_ids, position_ids, word_embeddings, position_embeddings):
    ids = jnp.transpose(input_ids.astype(jnp.int32), (1, 0)).reshape(NW, G, C)
    pids = jnp.transpose(position_ids.astype(jnp.int32), (1, 0)).reshape(
        NW, G, C)
    return _make_kernel()(ids, pids, word_embeddings, position_embeddings)


# speedup vs baseline: 1.4291x; 1.0690x over previous
import functools

import jax
import jax.numpy as jnp
from jax import lax
from jax.experimental import pallas as pl
from jax.experimental.pallas import tpu as pltpu
from jax.experimental.pallas import tpu_sc as plsc

VOCAB = 50257
HID = 2048
B = 4
S = 2048
NTOK = B * S

_info = plsc.get_sparse_core_info()
NC = _info.num_cores
NS = _info.num_subcores
NW = NC * NS
TPW = NTOK // NW
C = 8
G = TPW // C
VPR = HID // 16
NWB = 3
NOB = 4
PF = 2
UNROLL = 12


def _make_kernel():
    mesh = plsc.VectorSubcoreMesh(core_axis_name="c", subcore_axis_name="s")

    @functools.partial(
        pl.kernel,
        mesh=mesh,
        out_type=jax.ShapeDtypeStruct((S, B, HID), jnp.float32),
        scratch_types=[
            pltpu.VMEM((G, C), jnp.int32),
            pltpu.VMEM((G, C), jnp.int32),
        ] + [pltpu.VMEM((C, HID), jnp.float32)] * (NWB + NOB)
          + [pltpu.SemaphoreType.DMA] * (NWB + 2 * NOB),
    )
    def emb_kernel(wids_hbm, pids_hbm, wtab_hbm, ptab_hbm, out_hbm,
                   widx_v, pidx_v,
                   wbuf0, wbuf1, wbuf2,
                   obuf0, obuf1, obuf2, obuf3,
                   wsem0, wsem1, wsem2,
                   psem0, psem1, psem2, psem3,
                   osem0, osem1, osem2, osem3):
        wbufs = (wbuf0, wbuf1, wbuf2)
        obufs = (obuf0, obuf1, obuf2, obuf3)
        wsems = (wsem0, wsem1, wsem2)
        psems = (psem0, psem1, psem2, psem3)
        osems = (osem0, osem1, osem2, osem3)

        wid = lax.axis_index("s") * NC + lax.axis_index("c")
        sbase = wid * (TPW // B)
        pltpu.sync_copy(wids_hbm.at[wid], widx_v)
        pltpu.sync_copy(pids_hbm.at[wid], pidx_v)

        def start_gather(g, wb, ob):
            pltpu.async_copy(wtab_hbm.at[widx_v.at[g]], wbufs[wb], wsems[wb])
            pltpu.async_copy(ptab_hbm.at[pidx_v.at[g]], obufs[ob], psems[ob])

        def wait_gather(wb, ob):
            pltpu.make_async_copy(
                wtab_hbm.at[pl.ds(0, C)], wbufs[wb], wsems[wb]).wait()
            pltpu.make_async_copy(
                ptab_hbm.at[pl.ds(0, C)], obufs[ob], psems[ob]).wait()

        def wait_store(ob):
            for h in range(C // B):
                pltpu.make_async_copy(
                    obufs[ob].at[pl.ds(h * B, B)], out_hbm.at[0],
                    osems[ob]).wait()

        def do_add(wb, ob):
            def add_body(j, carry):
                col = j * 16
                for r in range(C):
                    obufs[ob][r, pl.ds(col, 16)] = (
                        obufs[ob][r, pl.ds(col, 16)]
                        + wbufs[wb][r, pl.ds(col, 16)]
                    )
                return carry
            lax.fori_loop(0, VPR, add_body, 0)

        def start_store(g, ob):
            for h in range(C // B):
                pltpu.async_copy(
                    obufs[ob].at[pl.ds(h * B, B)],
                    out_hbm.at[sbase + g * (C // B) + h],
                    osems[ob])

        def slot(g):
            wb = g % NWB
            ob = g % NOB
            if g + PF < G:
                if g >= PF:
                    wait_store((g + PF) % NOB)
                start_gather(g + PF, (g + PF) % NWB, (g + PF) % NOB)
            wait_gather(wb, ob)
            do_add(wb, ob)
            start_store(g, ob)

        for g in range(PF):
            start_gather(g, g % NWB, g % NOB)

        for g in range(UNROLL):
            slot(g)

        def step(i, carry):
            for b in range(UNROLL):
                g = i * UNROLL + b
                wait_store((b + PF) % NOB)
                start_gather(g + PF, (b + PF) % NWB, (b + PF) % NOB)
                wait_gather(b % NWB, b % NOB)
                do_add(b % NWB, b % NOB)
                start_store(g, b % NOB)
            return carry

        lax.fori_loop(1, (G - UNROLL) // UNROLL, step, 0)

        for g in range(UNROLL + UNROLL * ((G - UNROLL) // UNROLL - 1), G):
            slot(g)

        for g in range(G - PF, G):
            wait_store(g % NOB)

    return emb_kernel


def kernel(input_ids, position_ids, word_embeddings, position_embeddings):
    ids = jnp.transpose(input_ids.astype(jnp.int32), (1, 0)).reshape(NW, G, C)
    pids = jnp.transpose(position_ids.astype(jnp.int32), (1, 0)).reshape(
        NW, G, C)
    return _make_kernel()(ids, pids, word_embeddings, position_embeddings)
